# baseline (speedup 1.0000x reference)
import functools

import jax
import jax.numpy as jnp
from jax.experimental import pallas as pl
from jax.experimental.pallas import tpu as pltpu

C = 384
H = 224
W = 224
S = 16
NH = H // S
NW = W // S
K = NH * NW
G = 16
KG = (NH + 2) * G
WIN = 3 * G
PB = S * W
NEG = -1e30
INV_SQRT_C = float(1.0 / (C ** 0.5))
F32 = jnp.float32
BF16 = jnp.bfloat16


def _ssn_kernel(x_ref, poolw_ref, mask_ref, wk_ref, wv_ref,
                pix16_ref, ks_ref, vs_ref,
                cent_scr, num_scr, den_scr, prev_scr):
    i = pl.program_id(0)

    @pl.when(i == 0)
    def _():
        cent_scr[pl.ds(0, G), :] = jnp.zeros((G, C), F32)
        cent_scr[pl.ds(KG - G, G), :] = jnp.zeros((G, C), F32)
        num_scr[...] = jnp.zeros((KG, C), F32)
        den_scr[...] = jnp.zeros((KG, 128), F32)

    @pl.when(i < NH)
    def _():
        xb = x_ref[0]
        rowsum = jnp.sum(xb, axis=1)
        poolT = jax.lax.dot_general(
            poolw_ref[...], rowsum, (((0,), (1,)), ((), ())),
            preferred_element_type=F32)
        cent_scr[pl.ds((i + 1) * G, G), :] = poolT

    @pl.when(i >= 1)
    def _():
        bh = i - 1
        centw = cent_scr[pl.ds(bh * G, WIN), :]
        prev = prev_scr[...]
        dots = jax.lax.dot_general(
            centw.astype(BF16), prev, (((1,), (0,)), ((), ())),
            preferred_element_type=F32)
        csq = jnp.sum(centw * centw, axis=1, keepdims=True)
        r = jax.lax.broadcasted_iota(jnp.int32, (WIN, 1), 0)
        kh = bh - 1 + r // G
        pen = jnp.where((kh >= 0) & (kh < NH), 0.0, -NEG)
        lm = 2.0 * dots - (csq + pen) + mask_ref[...]
        m = jnp.max(lm, axis=0, keepdims=True)
        e = jnp.exp(lm - m)
        den = jnp.sum(e, axis=0, keepdims=True)
        aff = e / den
        contrib = jax.lax.dot_general(
            aff.astype(BF16), prev, (((1,), (1,)), ((), ())),
            preferred_element_type=F32)
        num_scr[pl.ds(bh * G, WIN), :] += contrib
        den_scr[pl.ds(bh * G, WIN), :] += jnp.broadcast_to(
            jnp.sum(aff, axis=1, keepdims=True), (WIN, 128))

    @pl.when(i < NH)
    def _():
        pixm = x_ref[0].astype(BF16).reshape(C, PB)
        pix16_ref[0] = pixm
        prev_scr[...] = pixm

    @pl.when(i == NH)
    def _():
        cent1 = (num_scr[...] /
                 (den_scr[...][:, :1] + 1e-16)).astype(BF16)
        ks_ref[...] = jax.lax.dot_general(
            cent1, wk_ref[...].astype(BF16), (((1,), (0,)), ((), ())),
            preferred_element_type=F32).astype(BF16)
        vs_ref[...] = jax.lax.dot_general(
            cent1, wv_ref[...].astype(BF16), (((1,), (0,)), ((), ())),
            preferred_element_type=F32).astype(BF16)


def _attn_kernel(pix_ref, wq_ref, wo_ref, ks_ref, vs_ref, y_ref):
    pixj = pix_ref[0]
    qT = jax.lax.dot_general(
        wq_ref[...].astype(BF16), pixj, (((0,), (0,)), ((), ())),
        preferred_element_type=F32)
    logits = jax.lax.dot_general(
        ks_ref[...], qT.astype(BF16), (((1,), (0,)), ((), ())),
        preferred_element_type=F32) * INV_SQRT_C
    r = jax.lax.broadcasted_iota(jnp.int32, (KG, 1), 0)
    colmask = jnp.where((r >= G) & (r < KG - G) & (r % G < NW), 0.0, NEG)
    lm = logits + colmask
    m = jnp.max(lm, axis=0, keepdims=True)
    e = jnp.exp(lm - m)
    attnT = e / jnp.sum(e, axis=0, keepdims=True)
    outT = jax.lax.dot_general(
        vs_ref[...], attnT.astype(BF16), (((0,), (0,)), ((), ())),
        preferred_element_type=F32)
    projT = jax.lax.dot_general(
        wo_ref[...].astype(BF16), outT.astype(BF16), (((0,), (0,)), ((), ())),
        preferred_element_type=F32)
    y_ref[0] = (pixj.astype(F32) + projT).reshape(C, S, W)


@functools.partial(jax.jit, static_argnames=("interpret",))
def kernel(x, Wq, Wk, Wv, Wo, interpret=False):
    poolw = (jnp.arange(G)[:, None] ==
             jnp.arange(W)[None, :] // S).astype(F32) / (S * S)
    poolw = poolw.T
    rr = jnp.arange(WIN)[:, None]
    ll = jnp.arange(PB)[None, :]
    kw = rr % G
    bw = (ll % W) // S
    maskadd = jnp.where((jnp.abs(kw - bw) <= 1) & (kw < NW), 0.0, NEG
                        ).astype(F32)

    pix16, ks16, vs16 = pl.pallas_call(
        _ssn_kernel,
        grid=(NH + 1,),
        in_specs=[
            pl.BlockSpec((1, C, S, W),
                         lambda i: (0, 0, jnp.minimum(i, NH - 1), 0)),
            pl.BlockSpec((W, G), lambda i: (0, 0)),
            pl.BlockSpec((WIN, PB), lambda i: (0, 0)),
            pl.BlockSpec((C, C), lambda i: (0, 0)),
            pl.BlockSpec((C, C), lambda i: (0, 0)),
        ],
        out_specs=[
            pl.BlockSpec((1, C, PB), lambda i: (jnp.minimum(i, NH - 1), 0, 0)),
            pl.BlockSpec((KG, C), lambda i: (0, 0)),
            pl.BlockSpec((KG, C), lambda i: (0, 0)),
        ],
        out_shape=[
            jax.ShapeDtypeStruct((NH, C, PB), BF16),
            jax.ShapeDtypeStruct((KG, C), BF16),
            jax.ShapeDtypeStruct((KG, C), BF16),
        ],
        scratch_shapes=[
            pltpu.VMEM((KG, C), F32),
            pltpu.VMEM((KG, C), F32),
            pltpu.VMEM((KG, 128), F32),
            pltpu.VMEM((C, PB), BF16),
        ],
        interpret=interpret,
    )(x, poolw, maskadd, Wk, Wv)

    y = pl.pallas_call(
        _attn_kernel,
        grid=(NH,),
        in_specs=[
            pl.BlockSpec((1, C, PB), lambda j: (j, 0, 0)),
            pl.BlockSpec((C, C), lambda j: (0, 0)),
            pl.BlockSpec((C, C), lambda j: (0, 0)),
            pl.BlockSpec((KG, C), lambda j: (0, 0)),
            pl.BlockSpec((KG, C), lambda j: (0, 0)),
        ],
        out_specs=pl.BlockSpec((1, C, S, W), lambda j: (0, 0, j, 0)),
        out_shape=jax.ShapeDtypeStruct((1, C, H, W), F32),
        compiler_params=pltpu.CompilerParams(
            dimension_semantics=("arbitrary",)),
        interpret=interpret,
    )(pix16, Wq, Wo, ks16, vs16)

    return y


# speedup vs baseline: 1.8884x; 1.8884x over previous
import functools

import jax
import jax.numpy as jnp
from jax.experimental import pallas as pl
from jax.experimental.pallas import tpu as pltpu

C = 384
H = 224
W = 224
S = 16
NH = H // S
NW = W // S
K = NH * NW
G = 16
KG = (NH + 2) * G
WIN = 3 * G
PB = S * W
NEG = -1e30
INV_SQRT_C = float(1.0 / (C ** 0.5))
F32 = jnp.float32
BF16 = jnp.bfloat16


def _ssn_kernel(x_ref, poolp_ref, mask_ref, ones_ref, wk_ref, wv_ref,
                pix16_ref, ks_ref, vs_ref,
                cent_scr, num_scr, den_scr, prev_scr):
    i = pl.program_id(0)

    @pl.when(i == 0)
    def _():
        cent_scr[pl.ds(0, G), :] = jnp.zeros((G, C), F32)
        cent_scr[pl.ds(KG - G, G), :] = jnp.zeros((G, C), F32)
        num_scr[...] = jnp.zeros((KG, C), F32)
        den_scr[...] = jnp.zeros((KG, 8), F32)

    @pl.when(i < NH)
    def _():
        xb = x_ref[0].reshape(PB, C)
        poolT = jax.lax.dot_general(
            poolp_ref[...], xb, (((0,), (0,)), ((), ())),
            preferred_element_type=F32)
        cent_scr[pl.ds((i + 1) * G, G), :] = poolT

    @pl.when(i >= 1)
    def _():
        bh = i - 1
        centw = cent_scr[pl.ds(bh * G, WIN), :]
        prev = prev_scr[...]
        dots = jax.lax.dot_general(
            prev, centw.astype(BF16), (((1,), (1,)), ((), ())),
            preferred_element_type=F32)
        csq = jnp.sum(centw * centw, axis=1)[None, :]
        r = jax.lax.broadcasted_iota(jnp.int32, (1, WIN), 1)
        kh = bh - 1 + r // G
        pen = jnp.where((kh >= 0) & (kh < NH), 0.0, -NEG)
        lm = 2.0 * dots - (csq + pen) + mask_ref[...]
        m = jnp.max(lm, axis=1, keepdims=True)
        e = jnp.exp(lm - m)
        den = jnp.sum(e, axis=1, keepdims=True)
        aff = (e / den).astype(BF16)
        contrib = jax.lax.dot_general(
            aff, prev, (((0,), (0,)), ((), ())),
            preferred_element_type=F32)
        dcon = jax.lax.dot_general(
            aff, ones_ref[...], (((0,), (0,)), ((), ())),
            preferred_element_type=F32)
        num_scr[pl.ds(bh * G, WIN), :] += contrib
        den_scr[pl.ds(bh * G, WIN), :] += dcon

    @pl.when(i < NH)
    def _():
        pixm = x_ref[0].astype(BF16).reshape(PB, C)
        pix16_ref[0] = pixm
        prev_scr[...] = pixm

    @pl.when(i == NH)
    def _():
        cent1 = (num_scr[...] /
                 (den_scr[...][:, :1] + 1e-16)).astype(BF16)
        ks_ref[...] = jax.lax.dot_general(
            cent1, wk_ref[...].astype(BF16), (((1,), (0,)), ((), ())),
            preferred_element_type=F32).astype(BF16)
        vs_ref[...] = jax.lax.dot_general(
            cent1, wv_ref[...].astype(BF16), (((1,), (0,)), ((), ())),
            preferred_element_type=F32).astype(BF16)


def _attn_kernel(pix_ref, wq_ref, wo_ref, ks_ref, vs_ref, y_ref):
    pixj = pix_ref[0]
    q = jax.lax.dot_general(
        pixj, wq_ref[...].astype(BF16), (((1,), (0,)), ((), ())),
        preferred_element_type=F32)
    logits = jax.lax.dot_general(
        q.astype(BF16), ks_ref[...], (((1,), (1,)), ((), ())),
        preferred_element_type=F32) * INV_SQRT_C
    r = jax.lax.broadcasted_iota(jnp.int32, (1, KG), 1)
    colmask = jnp.where((r >= G) & (r < KG - G) & (r % G < NW), 0.0, NEG)
    lm = logits + colmask
    m = jnp.max(lm, axis=1, keepdims=True)
    e = jnp.exp(lm - m)
    attn = (e / jnp.sum(e, axis=1, keepdims=True)).astype(BF16)
    out = jax.lax.dot_general(
        attn, vs_ref[...], (((1,), (0,)), ((), ())),
        preferred_element_type=F32)
    proj = jax.lax.dot_general(
        out.astype(BF16), wo_ref[...].astype(BF16), (((1,), (0,)), ((), ())),
        preferred_element_type=F32)
    y_ref[0] = (pixj.astype(F32) + proj).reshape(S, W, C)


@functools.partial(jax.jit, static_argnames=("interpret",))
def kernel(x, Wq, Wk, Wv, Wo, interpret=False):
    xt = x.transpose(0, 2, 3, 1)
    ll = jnp.arange(PB)[:, None]
    gg = jnp.arange(G)[None, :]
    poolp = ((ll % W) // S == gg).astype(F32) / (S * S)
    kw = jnp.arange(WIN)[None, :] % G
    bw = (jnp.arange(PB)[:, None] % W) // S
    maskadd = jnp.where((jnp.abs(kw - bw) <= 1) & (kw < NW), 0.0, NEG
                        ).astype(F32)
    ones8 = jnp.ones((PB, 8), BF16)

    pix16, ks16, vs16 = pl.pallas_call(
        _ssn_kernel,
        grid=(NH + 1,),
        in_specs=[
            pl.BlockSpec((1, S, W, C),
                         lambda i: (0, jnp.minimum(i, NH - 1), 0, 0)),
            pl.BlockSpec((PB, G), lambda i: (0, 0)),
            pl.BlockSpec((PB, WIN), lambda i: (0, 0)),
            pl.BlockSpec((PB, 8), lambda i: (0, 0)),
            pl.BlockSpec((C, C), lambda i: (0, 0)),
            pl.BlockSpec((C, C), lambda i: (0, 0)),
        ],
        out_specs=[
            pl.BlockSpec((1, PB, C), lambda i: (jnp.minimum(i, NH - 1), 0, 0)),
            pl.BlockSpec((KG, C), lambda i: (0, 0)),
            pl.BlockSpec((KG, C), lambda i: (0, 0)),
        ],
        out_shape=[
            jax.ShapeDtypeStruct((NH, PB, C), BF16),
            jax.ShapeDtypeStruct((KG, C), BF16),
            jax.ShapeDtypeStruct((KG, C), BF16),
        ],
        scratch_shapes=[
            pltpu.VMEM((KG, C), F32),
            pltpu.VMEM((KG, C), F32),
            pltpu.VMEM((KG, 8), F32),
            pltpu.VMEM((PB, C), BF16),
        ],
        interpret=interpret,
    )(xt, poolp, maskadd, ones8, Wk, Wv)

    y = pl.pallas_call(
        _attn_kernel,
        grid=(NH,),
        in_specs=[
            pl.BlockSpec((1, PB, C), lambda j: (j, 0, 0)),
            pl.BlockSpec((C, C), lambda j: (0, 0)),
            pl.BlockSpec((C, C), lambda j: (0, 0)),
            pl.BlockSpec((KG, C), lambda j: (0, 0)),
            pl.BlockSpec((KG, C), lambda j: (0, 0)),
        ],
        out_specs=pl.BlockSpec((1, S, W, C), lambda j: (0, j, 0, 0)),
        out_shape=jax.ShapeDtypeStruct((1, H, W, C), F32),
        compiler_params=pltpu.CompilerParams(
            dimension_semantics=("arbitrary",)),
        interpret=interpret,
    )(pix16, Wq, Wo, ks16, vs16)

    return y.transpose(0, 3, 1, 2)
